# baseline (speedup 1.0000x reference)
import jax
import jax.numpy as jnp
from jax import lax
from jax.experimental import pallas as pl
from jax.experimental.pallas import tpu as pltpu
from jax.experimental.pallas import tpu_sc as plsc

B, L, D = 1024, 200, 64
NC, NS = 2, 16
NW = NC * NS
NBT = B // 128
NITEM = L * NBT
PER_W = NITEM // NW
NRING = PER_W // 2


def _comb_body(seg_ref, pos_ref, out_ref):
    p = pos_ref[...]
    for s in range(3):
        out_ref[pl.ds(s * L, L), :] = p + seg_ref[s, :][None, :]


def _make_comb(seg_table, pos_l):
    return pl.pallas_call(
        _comb_body,
        out_shape=jax.ShapeDtypeStruct((3 * L, D), jnp.float32),
    )(seg_table, pos_l)


_FMT_SH = 14
_FMT_CH = 1 << _FMT_SH


def _fmt_body(a_ref, b_ref, o_ref):
    o_ref[:, pl.ds(0, 64)] = jnp.transpose(a_ref[...])
    o_ref[:, pl.ds(64, 64)] = jnp.transpose(b_ref[...])


def _fmt_table(table_t):
    grid = (1000000 + 2 * _FMT_CH - 1) // (2 * _FMT_CH)
    return pl.pallas_call(
        _fmt_body,
        grid=(grid,),
        in_specs=[
            pl.BlockSpec((64, _FMT_CH), lambda i: (0, 2 * i)),
            pl.BlockSpec(
                (64, _FMT_CH),
                lambda i: (0, jnp.minimum(2 * i + 1, 1000000 // _FMT_CH))),
        ],
        out_specs=pl.BlockSpec((_FMT_CH, 128), lambda i: (i, 0)),
        out_shape=jax.ShapeDtypeStruct((grid * _FMT_CH, 128), jnp.float32),
    )(table_t, table_t)


def _sc_body(x2, seg2, tbl2, comb, out, pidx, cidx,
             buf_a, buf_b, ostage, sem_a, sem_b, sem_o):
    wid = lax.axis_index("s") * NC + lax.axis_index("c")
    base_item = wid * PER_W
    iota16 = lax.iota(jnp.int32, 16)
    fck = [lax.shift_right_logical(iota16 + k * 16, 3) for k in range(4)]
    fi16 = iota16 & 7

    pltpu.sync_copy(x2.at[pl.ds(base_item, PER_W)], pidx)
    pltpu.sync_copy(seg2.at[pl.ds(base_item, PER_W)], cidx)

    def build(j, carry):
        l = (base_item + j) // NBT
        for g in range(8):
            sl = pl.ds(g * 16, 16)
            xv = pidx[j, sl]
            blk = lax.shift_right_logical(xv, _FMT_SH)
            off = xv & (_FMT_CH - 1)
            pidx[j, sl] = (
                (lax.shift_right_logical(blk, 1) * _FMT_CH + off) * 2
                + (blk & 1))
            cidx[j, sl] = cidx[j, sl] * L + l
        return carry

    lax.fori_loop(0, PER_W, build, 0)

    def fire(j, slot):
        pltpu.make_async_copy(
            tbl2.at[pidx.at[j]], buf_a.at[slot], sem_a.at[slot]).start()
        pltpu.make_async_copy(
            comb.at[cidx.at[j]], buf_b.at[slot], sem_b.at[slot]).start()

    def drain_out(i, slot):
        l, bt = i // NBT, i % NBT
        for fc in range(8):
            pltpu.make_async_copy(
                ostage.at[slot, fc, :, pl.ds(0, 128)],
                out.at[l, fc, bt], sem_o.at[slot]).wait()

    def compute(i, slot):
        l, bt = i // NBT, i % NBT
        ba, bb = buf_a.at[slot], buf_b.at[slot]
        os_ = ostage.at[slot]

        def per_tok(t0, carry):
            for dt in range(2):
                t = t0 * 2 + dt
                t16 = iota16 * 0 + t
                for k in range(4):
                    col = k * 16
                    v = ba[t, pl.ds(col, 16)] + bb[t, pl.ds(col, 16)]
                    plsc.store_scatter(os_, [fck[k], fi16, t16], v)
            return carry

        lax.fori_loop(0, 64, per_tok, 0)
        for fc in range(8):
            pltpu.make_async_copy(
                os_.at[fc, :, pl.ds(0, 128)], out.at[l, fc, bt],
                sem_o.at[slot]).start()

    fire(0, 0)
    fire(1, 1)

    def group(g2, carry):
        for slot in range(2):
            j = g2 * 2 + slot
            i = base_item + j
            pltpu.make_async_copy(
                tbl2.at[pidx.at[j]], buf_a.at[slot],
                sem_a.at[slot]).wait()
            pltpu.make_async_copy(
                comb.at[cidx.at[j]], buf_b.at[slot],
                sem_b.at[slot]).wait()

            @pl.when(g2 > 0)
            def _drain_prev():
                drain_out(i - 2, slot)

            compute(i, slot)

            @pl.when(g2 < NRING - 1)
            def _next():
                fire(j + 2, slot)
        return carry

    lax.fori_loop(0, NRING, group, 0)
    for slot in range(2):
        drain_out(base_item + PER_W - 2 + slot, slot)


_sc_call = pl.kernel(
    _sc_body,
    out_type=jax.ShapeDtypeStruct((L, 8, NBT, 8, 128), jnp.float32),
    mesh=plsc.VectorSubcoreMesh(
        core_axis_name="c", subcore_axis_name="s",
        num_cores=NC, num_subcores=NS),
    scratch_types=[
        pltpu.VMEM((PER_W, 128), jnp.int32),
        pltpu.VMEM((PER_W, 128), jnp.int32),
        pltpu.VMEM((2, 128, 64), jnp.float32),
        pltpu.VMEM((2, 128, 64), jnp.float32),
        pltpu.VMEM((2, 8, 8, 129), jnp.float32),
        pltpu.SemaphoreType.DMA((2,)),
        pltpu.SemaphoreType.DMA((2,)),
        pltpu.SemaphoreType.DMA((2,)),
    ],
    compiler_params=pltpu.CompilerParams(
        use_tc_tiling_on_sc=False, needs_layout_passes=False),
)


def kernel(x, segment_info, table, seg_table, pos):
    comb = _make_comb(seg_table, pos[:L])
    tbl2 = _fmt_table(table.T)
    tbl2 = tbl2.reshape(tbl2.shape[0] * 2, 64)
    x2 = x.astype(jnp.int32).T.reshape(NITEM, 128)
    s2 = segment_info.astype(jnp.int32).T.reshape(NITEM, 128)
    out5 = _sc_call(x2, s2, tbl2, comb)
    return out5.transpose(2, 4, 0, 1, 3).reshape(B, L, D)


# speedup vs baseline: 1.0060x; 1.0060x over previous
import jax
import jax.numpy as jnp
from jax import lax
from jax.experimental import pallas as pl
from jax.experimental.pallas import tpu as pltpu
from jax.experimental.pallas import tpu_sc as plsc

B, L, D = 1024, 200, 64
NC, NS = 2, 16
NW = NC * NS
NBT = B // 128
NITEM = L * NBT
PER_W = NITEM // NW
NSLOT = 5
NRING = PER_W // NSLOT


def _comb_body(seg_ref, pos_ref, out_ref):
    p = pos_ref[...]
    for s in range(3):
        out_ref[pl.ds(s * L, L), :] = p + seg_ref[s, :][None, :]


def _make_comb(seg_table, pos_l):
    return pl.pallas_call(
        _comb_body,
        out_shape=jax.ShapeDtypeStruct((3 * L, D), jnp.float32),
    )(seg_table, pos_l)


_FMT_SH = 14
_FMT_CH = 1 << _FMT_SH


def _fmt_body(a_ref, b_ref, o_ref):
    o_ref[:, pl.ds(0, 64)] = jnp.transpose(a_ref[...])
    o_ref[:, pl.ds(64, 64)] = jnp.transpose(b_ref[...])


def _fmt_table(table_t):
    grid = (1000000 + 2 * _FMT_CH - 1) // (2 * _FMT_CH)
    return pl.pallas_call(
        _fmt_body,
        grid=(grid,),
        in_specs=[
            pl.BlockSpec((64, _FMT_CH), lambda i: (0, 2 * i)),
            pl.BlockSpec(
                (64, _FMT_CH),
                lambda i: (0, jnp.minimum(2 * i + 1, 1000000 // _FMT_CH))),
        ],
        out_specs=pl.BlockSpec((_FMT_CH, 128), lambda i: (i, 0)),
        out_shape=jax.ShapeDtypeStruct((grid * _FMT_CH, 128), jnp.float32),
    )(table_t, table_t)


def _sc_body(x2, seg2, tbl2, comb, out, pidx, cidx,
             buf_a, buf_b, ostage, sem_a, sem_b, sem_o):
    wid = lax.axis_index("s") * NC + lax.axis_index("c")
    base_item = wid * PER_W
    iota16 = lax.iota(jnp.int32, 16)
    fck = [lax.shift_right_logical(iota16 + k * 16, 3) for k in range(4)]
    fi16 = iota16 & 7

    pltpu.sync_copy(x2.at[pl.ds(base_item, PER_W)], pidx)
    pltpu.sync_copy(seg2.at[pl.ds(base_item, PER_W)], cidx)

    def build(j, carry):
        l = (base_item + j) // NBT
        for g in range(8):
            sl = pl.ds(g * 16, 16)
            xv = pidx[j, sl]
            blk = lax.shift_right_logical(xv, _FMT_SH)
            off = xv & (_FMT_CH - 1)
            pidx[j, sl] = (
                (lax.shift_right_logical(blk, 1) * _FMT_CH + off) * 2
                + (blk & 1))
            cidx[j, sl] = cidx[j, sl] * L + l
        return carry

    lax.fori_loop(0, PER_W, build, 0)

    def fire(j, slot):
        pltpu.make_async_copy(
            tbl2.at[pidx.at[j]], buf_a.at[slot], sem_a.at[slot]).start()
        pltpu.make_async_copy(
            comb.at[cidx.at[j]], buf_b.at[slot], sem_b.at[slot]).start()

    def drain_out(i, slot):
        l, bt = i // NBT, i % NBT
        for fc in range(8):
            pltpu.make_async_copy(
                ostage.at[slot, fc, :, pl.ds(0, 128)],
                out.at[l, fc, bt], sem_o.at[slot]).wait()

    def compute(i, slot, oslot):
        l, bt = i // NBT, i % NBT
        ba, bb = buf_a.at[slot], buf_b.at[slot]
        os_ = ostage.at[oslot]

        def per_tok(t0, carry):
            for dt in range(2):
                t = t0 * 2 + dt
                t16 = iota16 * 0 + t
                for k in range(4):
                    col = k * 16
                    v = ba[t, pl.ds(col, 16)] + bb[t, pl.ds(col, 16)]
                    plsc.store_scatter(os_, [fck[k], fi16, t16], v)
            return carry

        lax.fori_loop(0, 64, per_tok, 0)
        for fc in range(8):
            pltpu.make_async_copy(
                os_.at[fc, :, pl.ds(0, 128)], out.at[l, fc, bt],
                sem_o.at[oslot]).start()

    OS = (0, 1, 0, 1, 0)
    PREV = (1, 3, 2, 2, 2)
    for b in range(NSLOT):
        fire(b, b)

    def group(g2, carry):
        for b in range(NSLOT):
            j = g2 * NSLOT + b
            i = base_item + j
            pltpu.make_async_copy(
                tbl2.at[pidx.at[j]], buf_a.at[b], sem_a.at[b]).wait()
            pltpu.make_async_copy(
                comb.at[cidx.at[j]], buf_b.at[b], sem_b.at[b]).wait()

            if b < 2:
                @pl.when(g2 > 0)
                def _drain_prev():
                    drain_out(i - PREV[b], OS[b])
            else:
                drain_out(i - PREV[b], OS[b])

            compute(i, b, OS[b])

            @pl.when(g2 < NRING - 1)
            def _next():
                fire(j + NSLOT, b)
        return carry

    lax.fori_loop(0, NRING, group, 0)
    drain_out(base_item + PER_W - 2, OS[3])
    drain_out(base_item + PER_W - 1, OS[4])


_sc_call = pl.kernel(
    _sc_body,
    out_type=jax.ShapeDtypeStruct((L, 8, NBT, 8, 128), jnp.float32),
    mesh=plsc.VectorSubcoreMesh(
        core_axis_name="c", subcore_axis_name="s",
        num_cores=NC, num_subcores=NS),
    scratch_types=[
        pltpu.VMEM((PER_W, 128), jnp.int32),
        pltpu.VMEM((PER_W, 128), jnp.int32),
        pltpu.VMEM((NSLOT, 128, 64), jnp.float32),
        pltpu.VMEM((NSLOT, 128, 64), jnp.float32),
        pltpu.VMEM((2, 8, 8, 129), jnp.float32),
        pltpu.SemaphoreType.DMA((NSLOT,)),
        pltpu.SemaphoreType.DMA((NSLOT,)),
        pltpu.SemaphoreType.DMA((2,)),
    ],
    compiler_params=pltpu.CompilerParams(
        use_tc_tiling_on_sc=False, needs_layout_passes=False),
)


def kernel(x, segment_info, table, seg_table, pos):
    comb = _make_comb(seg_table, pos[:L])
    tbl2 = _fmt_table(table.T)
    tbl2 = tbl2.reshape(tbl2.shape[0] * 2, 64)
    x2 = x.astype(jnp.int32).T.reshape(NITEM, 128)
    s2 = segment_info.astype(jnp.int32).T.reshape(NITEM, 128)
    out5 = _sc_call(x2, s2, tbl2, comb)
    return out5.transpose(2, 4, 0, 1, 3).reshape(B, L, D)
